# baseline (speedup 1.0000x reference)
import functools

import jax
import jax.numpy as jnp
from jax import lax
from jax.experimental import pallas as pl
from jax.experimental.pallas import tpu as pltpu
from jax.experimental.pallas import tpu_sc as plsc

B = 16384
E = 64
F = 128
U = 1000000
Q = 262144
CB = 16384
NC = 2
NS = 16
NW = NC * NS
BPW = B // NW

_n_colblocks = (U + CB - 1) // CB
_QB = Q // CB
_BB = (-(-U // (F * _QB)) + 7) // 8 * 8


def _pack_pair(lo_f32, hi_f32):
    lo16 = lax.bitcast_convert_type(lo_f32.astype(jnp.bfloat16), jnp.uint16)
    hi16 = lax.bitcast_convert_type(hi_f32.astype(jnp.bfloat16), jnp.uint16)
    word = lo16.astype(jnp.uint32) | (hi16.astype(jnp.uint32) << 16)
    return lax.bitcast_convert_type(word, jnp.float32)


def _xpose_body(c0_ref, c1_ref, c2_ref, c3_ref, b_ref, out_ref, pb_ref):
    i0 = lax.broadcasted_iota(jnp.int32, (E, E), 0)
    i1 = lax.broadcasted_iota(jnp.int32, (E, E), 1)
    ident = jnp.where(i0 == i1, 1.0, 0.0).astype(jnp.bfloat16)
    dn = (((0,), (0,)), ((), ()))

    def t(ref):
        return lax.dot_general(ref[...].astype(jnp.bfloat16), ident, dn,
                               preferred_element_type=jnp.float32)

    out_ref[:, :E] = _pack_pair(t(c0_ref), t(c1_ref))
    out_ref[:, E:] = _pack_pair(t(c2_ref), t(c3_ref))
    pb_ref[...] = b_ref[...].reshape(_BB, F)


def _in_spec(q):
    return pl.BlockSpec(
        (E, CB),
        lambda j, q=q: (0, jnp.minimum(j + q * _QB, _n_colblocks - 1)))


_tc_xpose = pl.pallas_call(
    _xpose_body,
    grid=(_QB,),
    compiler_params=pltpu.CompilerParams(dimension_semantics=("parallel",)),
    in_specs=[_in_spec(0), _in_spec(1), _in_spec(2), _in_spec(3),
              pl.BlockSpec((1, _BB * F), lambda j: (0, j))],
    out_specs=[pl.BlockSpec((CB, 2 * E), lambda j: (j, 0)),
               pl.BlockSpec((_BB, F), lambda j: (j, 0))],
    out_shape=(jax.ShapeDtypeStruct((Q, 2 * E), jnp.float32),
               jax.ShapeDtypeStruct((_QB * _BB, F), jnp.float32)),
)

_sc_mesh = plsc.VectorSubcoreMesh(core_axis_name="c", subcore_axis_name="s")


@functools.partial(
    pl.kernel,
    mesh=_sc_mesh,
    compiler_params=pltpu.CompilerParams(needs_layout_passes=False),
    out_type=(
        jax.ShapeDtypeStruct((B, 2 * E), jnp.float32),
        jax.ShapeDtypeStruct((B,), jnp.float32),
    ),
    scratch_types=[
        pltpu.VMEM((BPW,), jnp.int32),
        pltpu.VMEM((BPW,), jnp.int32),
        pltpu.VMEM((BPW,), jnp.int32),
        pltpu.VMEM((BPW, 2 * E), jnp.float32),
        pltpu.VMEM((BPW // 2, F), jnp.float32),
        pltpu.VMEM((BPW,), jnp.float32),
        pltpu.SemaphoreType.DMA,
        pltpu.SemaphoreType.DMA,
    ],
)
def _sc_gather(p_hbm, pbias_hbm, pidx_hbm, idx_hbm, bidx_hbm,
               rows_out, bias_out,
               pidx_v, idx_v, bidx_v, rows_v, brows_v, bias_v, sem_e, sem_b):
    wid = lax.axis_index("s") * NC + lax.axis_index("c")
    base = wid * BPW
    half = BPW // 2
    pltpu.sync_copy(pidx_hbm.at[pl.ds(base, BPW)], pidx_v)
    pltpu.sync_copy(idx_hbm.at[pl.ds(base, BPW)], idx_v)
    pltpu.sync_copy(bidx_hbm.at[pl.ds(base, BPW)], bidx_v)
    ce = pltpu.async_copy(p_hbm.at[pidx_v], rows_v, sem_e)
    for c in range(2):
        pltpu.async_copy(pbias_hbm.at[bidx_v.at[pl.ds(c * half, half)]],
                         brows_v, sem_b).wait()

        @pl.loop(0, half, step=16)
        def _(r, c=c):
            sel = idx_v[pl.ds(c * half + r, 16)] & (F - 1)
            rows16 = lax.iota(jnp.int32, 16) + r
            bias_v[pl.ds(c * half + r, 16)] = plsc.load_gather(
                brows_v, [rows16, sel])

    ce.wait()
    pltpu.sync_copy(rows_v, rows_out.at[pl.ds(base, BPW)])
    pltpu.sync_copy(bias_v, bias_out.at[pl.ds(base, BPW)])


BLK = 4096


def _unpack(word_f32, hi):
    u = lax.bitcast_convert_type(word_f32, jnp.uint32)
    h = jnp.where(hi, u >> 16, u & 0xFFFF).astype(jnp.uint16)
    return lax.bitcast_convert_type(h, jnp.bfloat16).astype(jnp.float32)


def _tc_proj_body(feats_ref, w_ref, b_ref, proj_ref):
    proj_ref[...] = jnp.dot(feats_ref[...], w_ref[...],
                            preferred_element_type=jnp.float32) + b_ref[...]


_tc_proj = pl.pallas_call(
    _tc_proj_body,
    grid=(B // BLK,),
    in_specs=[
        pl.BlockSpec((BLK, F), lambda i: (i, 0)),
        pl.BlockSpec((F, E), lambda i: (0, 0)),
        pl.BlockSpec((1, E), lambda i: (0, 0)),
    ],
    out_specs=pl.BlockSpec((BLK, E), lambda i: (i, 0)),
    out_shape=jax.ShapeDtypeStruct((B, E), jnp.float32),
)


def _tc_body(proj_ref, rows_ref, bias_ref, idx_ref, out_ref):
    proj = proj_ref[...]
    d = []
    for half in (rows_ref[:, :E], rows_ref[:, E:]):
        for hi in (False, True):
            emb = _unpack(half, hi)
            d.append(jnp.sum(emb * proj, axis=1, keepdims=True).T)
    quarter = idx_ref[...] // Q
    dq = jnp.where(quarter >= 2,
                   jnp.where(quarter == 3, d[3], d[2]),
                   jnp.where(quarter == 1, d[1], d[0]))
    out_ref[...] = jax.nn.sigmoid(dq + bias_ref[...])


_tc_combine = pl.pallas_call(
    _tc_body,
    grid=(B // BLK,),
    compiler_params=pltpu.CompilerParams(dimension_semantics=("parallel",)),
    in_specs=[
        pl.BlockSpec((BLK, E), lambda i: (i, 0)),
        pl.BlockSpec((BLK, 2 * E), lambda i: (i, 0)),
        pl.BlockSpec((1, BLK), lambda i: (0, i)),
        pl.BlockSpec((1, BLK), lambda i: (0, i)),
    ],
    out_specs=pl.BlockSpec((1, BLK), lambda i: (0, i)),
    out_shape=jax.ShapeDtypeStruct((1, B), jnp.float32),
)


def kernel(user_ids, restaurant_features, user_emb, user_bias_table,
           dense_W, dense_b):
    idx = user_ids.astype(jnp.int32).reshape(B)
    pidx = idx % Q
    bidx = idx >> 7
    embt = user_emb.T
    bias_row = user_bias_table.T
    p, pbias = _tc_xpose(embt, embt, embt, embt, bias_row)
    rows_g, bias_g = _sc_gather(p, pbias, pidx, idx, bidx)
    proj = _tc_proj(restaurant_features, dense_W, dense_b.reshape(1, E))
    out = _tc_combine(proj, rows_g, bias_g.reshape(1, B), idx.reshape(1, B))
    return out.reshape(B, 1)


# speedup vs baseline: 1.0236x; 1.0236x over previous
import functools

import jax
import jax.numpy as jnp
from jax import lax
from jax.experimental import pallas as pl
from jax.experimental.pallas import tpu as pltpu
from jax.experimental.pallas import tpu_sc as plsc

B = 16384
E = 64
F = 128
U = 1000000
Q = 262144
CB = 16384
NC = 2
NS = 16
NW = NC * NS
BPW = B // NW

_n_colblocks = (U + CB - 1) // CB
_QB = Q // CB
_BB = (-(-U // (F * _QB)) + 7) // 8 * 8


def _pack_pair(lo_f32, hi_f32):
    lo16 = lax.bitcast_convert_type(lo_f32.astype(jnp.bfloat16), jnp.uint16)
    hi16 = lax.bitcast_convert_type(hi_f32.astype(jnp.bfloat16), jnp.uint16)
    word = lo16.astype(jnp.uint32) | (hi16.astype(jnp.uint32) << 16)
    return lax.bitcast_convert_type(word, jnp.float32)


def _xpose_body(c0_ref, c1_ref, c2_ref, c3_ref, b_ref, out_ref, pb_ref):
    i0 = lax.broadcasted_iota(jnp.int32, (E, E), 0)
    i1 = lax.broadcasted_iota(jnp.int32, (E, E), 1)
    ident = jnp.where(i0 == i1, 1.0, 0.0).astype(jnp.bfloat16)
    dn = (((0,), (0,)), ((), ()))

    def t(ref):
        return lax.dot_general(ref[...].astype(jnp.bfloat16), ident, dn,
                               preferred_element_type=jnp.float32)

    out_ref[:, :E] = _pack_pair(t(c0_ref), t(c1_ref))
    out_ref[:, E:] = _pack_pair(t(c2_ref), t(c3_ref))
    pb_ref[...] = b_ref[...].reshape(_BB, F)


def _in_spec(q):
    return pl.BlockSpec(
        (E, CB),
        lambda j, q=q: (0, jnp.minimum(j + q * _QB, _n_colblocks - 1)))


_tc_xpose = pl.pallas_call(
    _xpose_body,
    grid=(_QB,),
    compiler_params=pltpu.CompilerParams(dimension_semantics=("parallel",)),
    in_specs=[_in_spec(0), _in_spec(1), _in_spec(2), _in_spec(3),
              pl.BlockSpec((1, _BB * F), lambda j: (0, j))],
    out_specs=[pl.BlockSpec((CB, 2 * E), lambda j: (j, 0)),
               pl.BlockSpec((_BB, F), lambda j: (j, 0))],
    out_shape=(jax.ShapeDtypeStruct((Q, 2 * E), jnp.float32),
               jax.ShapeDtypeStruct((_QB * _BB, F), jnp.float32)),
)

_sc_mesh = plsc.VectorSubcoreMesh(core_axis_name="c", subcore_axis_name="s")


@functools.partial(
    pl.kernel,
    mesh=_sc_mesh,
    compiler_params=pltpu.CompilerParams(needs_layout_passes=False),
    out_type=(
        jax.ShapeDtypeStruct((B, 2 * E), jnp.float32),
        jax.ShapeDtypeStruct((B,), jnp.float32),
    ),
    scratch_types=[
        pltpu.VMEM((BPW,), jnp.int32),
        pltpu.VMEM((BPW,), jnp.int32),
        pltpu.VMEM((BPW,), jnp.int32),
        pltpu.VMEM((BPW, 2 * E), jnp.float32),
        pltpu.VMEM((BPW // 2, F), jnp.float32),
        pltpu.VMEM((BPW,), jnp.float32),
        pltpu.SemaphoreType.DMA,
        pltpu.SemaphoreType.DMA,
    ],
)
def _sc_gather(p_hbm, pbias_hbm, pidx_hbm, idx_hbm, bidx_hbm,
               rows_out, bias_out,
               pidx_v, idx_v, bidx_v, rows_v, brows_v, bias_v, sem_e, sem_b):
    wid = lax.axis_index("s") * NC + lax.axis_index("c")
    base = wid * BPW
    half = BPW // 2
    pltpu.sync_copy(pidx_hbm.at[pl.ds(base, BPW)], pidx_v)
    pltpu.sync_copy(idx_hbm.at[pl.ds(base, BPW)], idx_v)
    pltpu.sync_copy(bidx_hbm.at[pl.ds(base, BPW)], bidx_v)
    ce = pltpu.async_copy(p_hbm.at[pidx_v], rows_v, sem_e)
    for c in range(2):
        pltpu.async_copy(pbias_hbm.at[bidx_v.at[pl.ds(c * half, half)]],
                         brows_v, sem_b).wait()

        @pl.loop(0, half, step=16)
        def _(r, c=c):
            sel = idx_v[pl.ds(c * half + r, 16)] & (F - 1)
            rows16 = lax.iota(jnp.int32, 16) + r
            bias_v[pl.ds(c * half + r, 16)] = plsc.load_gather(
                brows_v, [rows16, sel])

    ce.wait()
    pltpu.sync_copy(rows_v, rows_out.at[pl.ds(base, BPW)])
    pltpu.sync_copy(bias_v, bias_out.at[pl.ds(base, BPW)])


BLK = 4096


def _unpack(word_f32, hi):
    u = lax.bitcast_convert_type(word_f32, jnp.uint32)
    h = jnp.where(hi, u >> 16, u & 0xFFFF).astype(jnp.uint16)
    return lax.bitcast_convert_type(h, jnp.bfloat16).astype(jnp.float32)


def _tc_body(feats_ref, w_ref, b_ref, rows_ref, bias_ref, idx_ref, out_ref):
    proj = jnp.dot(feats_ref[...], w_ref[...],
                   preferred_element_type=jnp.float32) + b_ref[...]
    d = []
    for half in (rows_ref[:, :E], rows_ref[:, E:]):
        for hi in (False, True):
            emb = _unpack(half, hi)
            d.append(jnp.sum(emb * proj, axis=1, keepdims=True).T)
    quarter = idx_ref[...] // Q
    dq = jnp.where(quarter >= 2,
                   jnp.where(quarter == 3, d[3], d[2]),
                   jnp.where(quarter == 1, d[1], d[0]))
    out_ref[...] = jax.nn.sigmoid(dq + bias_ref[...])


_tc_combine = pl.pallas_call(
    _tc_body,
    grid=(B // BLK,),
    compiler_params=pltpu.CompilerParams(dimension_semantics=("parallel",)),
    in_specs=[
        pl.BlockSpec((BLK, F), lambda i: (i, 0)),
        pl.BlockSpec((F, E), lambda i: (0, 0)),
        pl.BlockSpec((1, E), lambda i: (0, 0)),
        pl.BlockSpec((BLK, 2 * E), lambda i: (i, 0)),
        pl.BlockSpec((1, BLK), lambda i: (0, i)),
        pl.BlockSpec((1, BLK), lambda i: (0, i)),
    ],
    out_specs=pl.BlockSpec((1, BLK), lambda i: (0, i)),
    out_shape=jax.ShapeDtypeStruct((1, B), jnp.float32),
)


def kernel(user_ids, restaurant_features, user_emb, user_bias_table,
           dense_W, dense_b):
    idx = user_ids.astype(jnp.int32).reshape(B)
    pidx = idx % Q
    bidx = idx >> 7
    embt = user_emb.T
    bias_row = user_bias_table.T
    p, pbias = _tc_xpose(embt, embt, embt, embt, bias_row)
    rows_g, bias_g = _sc_gather(p, pbias, pidx, idx, bidx)
    out = _tc_combine(restaurant_features, dense_W, dense_b.reshape(1, E),
                      rows_g, bias_g.reshape(1, B), idx.reshape(1, B))
    return out.reshape(B, 1)
